# baseline (speedup 1.0000x reference)
import functools

import jax
import jax.numpy as jnp
from jax import lax
from jax.experimental import pallas as pl
from jax.experimental.pallas import tpu as pltpu
from jax.experimental.pallas import tpu_sc as plsc

HIDDEN = 32
NC = 2
NS = 16
NW = NC * NS
NB = 16384
SEQ = 26
B = NB * SEQ
BT = NB // 128
TPW = BT // NW
B_PER_W = B // NW
STEPS = TPW * SEQ

_mesh = plsc.VectorSubcoreMesh(core_axis_name="c", subcore_axis_name="s")


@functools.partial(
    pl.kernel,
    mesh=_mesh,
    out_type=jax.ShapeDtypeStruct((SEQ, HIDDEN // 8, BT, 8, 128), jnp.float32),
    scratch_types=[
        pltpu.VMEM((B_PER_W,), jnp.int32),
        pltpu.VMEM((B_PER_W,), jnp.int32),
        [pltpu.VMEM((128, HIDDEN), jnp.float32) for _ in range(2)],
        [pltpu.VMEM((HIDDEN // 8, 8, 128), jnp.float32) for _ in range(2)],
        [pltpu.SemaphoreType.DMA for _ in range(2)],
        [pltpu.SemaphoreType.DMA for _ in range(2)],
    ],
    compiler_params=pltpu.CompilerParams(use_tc_tiling_on_sc=False, needs_layout_passes=False),
)
def _gather_all(idx_hbm, w_hbm, out_hbm, idx_raw, idx2, rows, shuf, gsem, ssem):
    wid = lax.axis_index("s") * NC + lax.axis_index("c")
    base = wid * B_PER_W
    iota = lax.iota(jnp.int32, 16)

    pltpu.sync_copy(idx_hbm.at[pl.ds(base, B_PER_W)], idx_raw)

    def perm_body(g, carry):
        t = g // (SEQ * 8)
        rem = g % (SEQ * 8)
        s = rem // 8
        b0 = (rem % 8) * 16
        src = SEQ * (t * 128 + b0) + s + SEQ * iota
        v = plsc.load_gather(idx_raw, [src])
        idx2[pl.ds((t * SEQ + s) * 128 + b0, 16)] = v
        return carry

    lax.fori_loop(0, STEPS * 8, perm_body, 0)

    def start_gather(k, par):
        pltpu.make_async_copy(
            w_hbm.at[idx2.at[pl.ds(k * 128, 128)]], rows[par], gsem[par]
        ).start()

    def wait_gather(par):
        pltpu.make_async_copy(
            w_hbm.at[idx2.at[pl.ds(0, 128)]], rows[par], gsem[par]).wait()

    def store_block(k, par):
        t = k // SEQ
        s = k % SEQ
        bt = wid * TPW + t
        pltpu.make_async_copy(shuf[par], out_hbm.at[s, :, bt], ssem[par]).start()

    def wait_store(par):
        pltpu.make_async_copy(out_hbm.at[0, :, 0], shuf[par], ssem[par]).wait()

    def shuffle(par):
        def shuf_body(h, carry):
            ht = h // 8
            hs = h % 8
            hvec = jnp.full((16,), h, jnp.int32)
            for g in range(8):
                b0 = g * 16
                v = plsc.load_gather(rows[par], [b0 + iota, hvec])
                shuf[par][ht, hs, pl.ds(b0, 16)] = v
            return carry

        lax.fori_loop(0, HIDDEN, shuf_body, 0)

    start_gather(0, 0)
    start_gather(1, 1)

    def step(k, par):
        wait_gather(par)

        @pl.when(k >= 2)
        def _():
            wait_store(par)

        shuffle(par)

        @pl.when(k + 2 < STEPS)
        def _():
            start_gather(k + 2, par)

        store_block(k, par)

    def loop_body(i, carry):
        step(2 * i, 0)
        step(2 * i + 1, 1)
        return carry

    lax.fori_loop(0, STEPS // 2, loop_body, 0)
    wait_store(0)
    wait_store(1)


@jax.jit
def kernel(x, weight):
    flat = x.reshape(-1).astype(jnp.int32)
    out5 = _gather_all(flat, weight)
    return out5.transpose((2, 4, 0, 1, 3)).reshape(NB, SEQ, HIDDEN)
